# baseline (speedup 1.0000x reference)
import functools

import jax
import jax.numpy as jnp
import numpy as np
from jax import lax
from jax.experimental import pallas as pl
from jax.experimental.pallas import tpu as pltpu
from jax.experimental.pallas import tpu_sc as plsc

BATCH = 4096
VOCAB = 8192
SEQ = 200
D = 64
NC = 2
NS = 16
NW = NC * NS
BPW = BATCH // NW
L = 16
NG = D // L
G0 = 128
G1 = SEQ - G0

_PERM = np.array(
    [h * 32 + off + i for h in range(D // 32) for i in range(L) for off in (0, L)],
    dtype=np.int32,
)

_mesh = plsc.VectorSubcoreMesh(
    core_axis_name="c", subcore_axis_name="s", num_cores=NC, num_subcores=NS
)


@functools.partial(
    pl.kernel,
    out_type=jax.ShapeDtypeStruct((BATCH, SEQ, D), jnp.float32),
    mesh=_mesh,
    scratch_types=[
        pltpu.VMEM((SEQ, D), jnp.float32),
        pltpu.VMEM((BPW,), jnp.int32),
        pltpu.VMEM((BPW, D), jnp.float32),
        pltpu.VMEM((BPW, SEQ), jnp.int32),
        pltpu.VMEM((2, SEQ, D // 2), jnp.int32),
        pltpu.VMEM((2, SEQ, D), jnp.float32),
        pltpu.VMEM_SHARED((VOCAB, D // 2), jnp.int32),
        pltpu.SemaphoreType.DMA,
        pltpu.SemaphoreType.DMA,
        pltpu.SemaphoreType.DMA,
        pltpu.SemaphoreType.DMA,
        pltpu.SemaphoreType.DMA,
    ],
    compiler_params=pltpu.CompilerParams(use_tc_tiling_on_sc=False),
)
def _embed(x_hbm, cond_hbm, lat_hbm, ct_hbm, pe_hbm, out_hbm,
           pe_v, ci_v, cr_v, idx_v, rows_v, ob_v, lat_s,
           gsem0, gsem1, osem0, osem1, csem):
    wid = lax.axis_index("s") * NC + lax.axis_index("c")
    base = wid * BPW
    gsems = (gsem0, gsem1)
    osems = (osem0, osem1)

    def gather_descs(buf, j):
        d0 = pltpu.make_async_copy(
            lat_s.at[idx_v.at[j, pl.ds(0, G0)]],
            rows_v.at[buf, pl.ds(0, G0)], gsems[buf])
        d1 = pltpu.make_async_copy(
            lat_s.at[idx_v.at[j, pl.ds(G0, G1)]],
            rows_v.at[buf, pl.ds(G0, G1)], gsems[buf])
        return d0, d1

    def start_gather(buf, j):
        d0, d1 = gather_descs(buf, j)
        d0.start()
        d1.start()

    def wait_gather(buf, j):
        d0, d1 = gather_descs(buf, j)
        d0.wait()
        d1.wait()

    def out_desc(buf, j):
        return pltpu.make_async_copy(ob_v.at[buf], out_hbm.at[base + j], osems[buf])

    def compute(buf, j):
        c = [cr_v[j, pl.ds(g * L, L)] for g in range(NG)]

        @pl.loop(0, SEQ)
        def _(s):
            for h in range(D // 32):
                w = rows_v[buf, s, pl.ds(h * L, L)]
                a = lax.bitcast_convert_type(lax.shift_left(w, 16), jnp.float32)
                b = lax.bitcast_convert_type(
                    lax.bitwise_and(w, jnp.int32(-65536)), jnp.float32)
                ga, gb = 2 * h, 2 * h + 1
                ob_v[buf, s, pl.ds(ga * L, L)] = a + pe_v[s, pl.ds(ga * L, L)] + c[ga]
                ob_v[buf, s, pl.ds(gb * L, L)] = b + pe_v[s, pl.ds(gb * L, L)] + c[gb]

    sid = lax.axis_index("s")
    rpt = VOCAB // NS
    pltpu.sync_copy(lat_hbm.at[pl.ds(sid * rpt, rpt)],
                    lat_s.at[pl.ds(sid * rpt, rpt)])
    plsc.subcore_barrier()

    pltpu.sync_copy(pe_hbm, pe_v)
    pltpu.sync_copy(x_hbm.at[pl.ds(base, BPW)], idx_v)
    pltpu.sync_copy(cond_hbm.at[pl.ds(base, BPW)], ci_v)
    pltpu.async_copy(ct_hbm.at[ci_v], cr_v, csem).wait()

    start_gather(0, 0)

    @pl.loop(0, BPW // 2)
    def _(i):
        j0 = 2 * i

        start_gather(1, j0 + 1)
        wait_gather(0, j0)

        @pl.when(i >= 1)
        def _():
            out_desc(0, j0 - 2).wait()
        compute(0, j0)
        out_desc(0, j0).start()

        @pl.when(i < BPW // 2 - 1)
        def _():
            start_gather(0, j0 + 2)
        wait_gather(1, j0 + 1)

        @pl.when(i >= 1)
        def _():
            out_desc(1, j0 - 1).wait()
        compute(1, j0 + 1)
        out_desc(1, j0 + 1).start()

    out_desc(0, BPW - 2).wait()
    out_desc(1, BPW - 1).wait()


def kernel(x, cond, latent_table, cond_table, pe):
    lat_bf = latent_table[:, _PERM].astype(jnp.bfloat16)
    lat_pairs = lax.bitcast_convert_type(
        lat_bf.reshape(VOCAB, D // 2, 2), jnp.int32)
    return _embed(x, cond, lat_pairs, cond_table, pe[:SEQ])


# speedup vs baseline: 1.0105x; 1.0105x over previous
import functools

import jax
import jax.numpy as jnp
import numpy as np
from jax import lax
from jax.experimental import pallas as pl
from jax.experimental.pallas import tpu as pltpu
from jax.experimental.pallas import tpu_sc as plsc

BATCH = 4096
VOCAB = 8192
SEQ = 200
D = 64
NC = 2
NS = 16
NW = NC * NS
BPW = BATCH // NW
L = 16
NG = D // L
G0 = 128
G1 = SEQ - G0

_mesh = plsc.VectorSubcoreMesh(
    core_axis_name="c", subcore_axis_name="s", num_cores=NC, num_subcores=NS
)


@functools.partial(
    pl.kernel,
    out_type=jax.ShapeDtypeStruct((BATCH, SEQ, D), jnp.float32),
    mesh=_mesh,
    scratch_types=[
        pltpu.VMEM((SEQ, D), jnp.float32),
        pltpu.VMEM((BPW,), jnp.int32),
        pltpu.VMEM((BPW, D), jnp.float32),
        pltpu.VMEM((BPW, SEQ), jnp.int32),
        pltpu.VMEM((2, SEQ, D // 2), jnp.int32),
        pltpu.VMEM((2, SEQ, D), jnp.float32),
        pltpu.VMEM_SHARED((VOCAB, D // 2), jnp.int32),
        pltpu.SemaphoreType.DMA,
        pltpu.SemaphoreType.DMA,
        pltpu.SemaphoreType.DMA,
        pltpu.SemaphoreType.DMA,
        pltpu.SemaphoreType.DMA,
    ],
    compiler_params=pltpu.CompilerParams(use_tc_tiling_on_sc=False),
)
def _embed(x_hbm, cond_hbm, lat_hbm, ct_hbm, pe_hbm, out_hbm,
           pe_v, ci_v, cr_v, idx_v, rows_v, ob_v, lat_s,
           gsem0, gsem1, osem0, osem1, csem):
    wid = lax.axis_index("s") * NC + lax.axis_index("c")
    base = wid * BPW
    gsems = (gsem0, gsem1)
    osems = (osem0, osem1)

    def gather_descs(buf, j):
        d0 = pltpu.make_async_copy(
            lat_s.at[idx_v.at[j, pl.ds(0, G0)]],
            rows_v.at[buf, pl.ds(0, G0)], gsems[buf])
        d1 = pltpu.make_async_copy(
            lat_s.at[idx_v.at[j, pl.ds(G0, G1)]],
            rows_v.at[buf, pl.ds(G0, G1)], gsems[buf])
        return d0, d1

    def start_gather(buf, j):
        d0, d1 = gather_descs(buf, j)
        d0.start()
        d1.start()

    def wait_gather(buf, j):
        d0, d1 = gather_descs(buf, j)
        d0.wait()
        d1.wait()

    def out_desc(buf, j):
        return pltpu.make_async_copy(ob_v.at[buf], out_hbm.at[base + j], osems[buf])

    def compute(buf, j):
        c = [cr_v[j, pl.ds(g * L, L)] for g in range(NG)]

        @pl.loop(0, SEQ)
        def _(s):
            for h in range(D // 32):
                w = rows_v[buf, s, pl.ds(h * L, L)]
                a = lax.bitcast_convert_type(lax.shift_left(w, 16), jnp.float32)
                b = lax.bitcast_convert_type(
                    lax.bitwise_and(w, jnp.int32(-65536)), jnp.float32)
                ga, gb = 2 * h, 2 * h + 1
                ob_v[buf, s, pl.ds(ga * L, L)] = a + pe_v[s, pl.ds(ga * L, L)] + c[ga]
                ob_v[buf, s, pl.ds(gb * L, L)] = b + pe_v[s, pl.ds(gb * L, L)] + c[gb]

    sid = lax.axis_index("s")
    rpt = VOCAB // NS
    pltpu.sync_copy(lat_hbm.at[pl.ds(sid * rpt, rpt)],
                    lat_s.at[pl.ds(sid * rpt, rpt)])
    plsc.subcore_barrier()

    pltpu.sync_copy(pe_hbm, pe_v)
    pltpu.sync_copy(x_hbm.at[pl.ds(base, BPW)], idx_v)
    pltpu.sync_copy(cond_hbm.at[pl.ds(base, BPW)], ci_v)
    pltpu.async_copy(ct_hbm.at[ci_v], cr_v, csem).wait()

    start_gather(0, 0)

    @pl.loop(0, BPW // 2)
    def _(i):
        j0 = 2 * i

        start_gather(1, j0 + 1)
        wait_gather(0, j0)

        @pl.when(i >= 1)
        def _():
            out_desc(0, j0 - 2).wait()
        compute(0, j0)
        out_desc(0, j0).start()

        @pl.when(i < BPW // 2 - 1)
        def _():
            start_gather(0, j0 + 2)
        wait_gather(1, j0 + 1)

        @pl.when(i >= 1)
        def _():
            out_desc(1, j0 - 1).wait()
        compute(1, j0 + 1)
        out_desc(1, j0 + 1).start()

    out_desc(0, BPW - 2).wait()
    out_desc(1, BPW - 1).wait()


def kernel(x, cond, latent_table, cond_table, pe):
    lat_bf = (latent_table.astype(jnp.bfloat16)
              .reshape(VOCAB, D // 32, 2, L)
              .transpose(0, 1, 3, 2))
    lat_pairs = lax.bitcast_convert_type(lat_bf, jnp.int32).reshape(VOCAB, D // 2)
    return _embed(x, cond, lat_pairs, cond_table, pe[:SEQ])


# speedup vs baseline: 1.3961x; 1.3815x over previous
import functools

import jax
import jax.numpy as jnp
from jax import lax
from jax.experimental import pallas as pl
from jax.experimental.pallas import tpu as pltpu
from jax.experimental.pallas import tpu_sc as plsc

BATCH = 4096
VOCAB = 8192
SEQ = 200
D = 64
NC = 2
NS = 16
NW = NC * NS
BPW = BATCH // NW
L = 16
NG = D // L

_CHUNKS = tuple(16 * k for k in range(SEQ // 16)) + (SEQ - 16,)

_mesh = plsc.VectorSubcoreMesh(
    core_axis_name="c", subcore_axis_name="s", num_cores=NC, num_subcores=NS
)


@functools.partial(
    pl.kernel,
    out_type=jax.ShapeDtypeStruct((BATCH, SEQ, D), jnp.float32),
    mesh=_mesh,
    scratch_types=[
        pltpu.VMEM((SEQ, D), jnp.float32),
        pltpu.VMEM((BPW,), jnp.int32),
        pltpu.VMEM((BPW, D), jnp.float32),
        pltpu.VMEM((BPW, SEQ), jnp.int32),
        pltpu.VMEM((2, SEQ, D), jnp.float32),
        pltpu.VMEM((2, SEQ, D), jnp.float32),
        pltpu.VMEM_SHARED((VOCAB, D), jnp.float32),
        pltpu.SemaphoreType.DMA,
        pltpu.SemaphoreType.DMA,
        pltpu.SemaphoreType.DMA,
        pltpu.SemaphoreType.DMA,
        pltpu.SemaphoreType.DMA,
    ],
    compiler_params=pltpu.CompilerParams(use_tc_tiling_on_sc=False),
)
def _embed(x_hbm, cond_hbm, lat_hbm, ct_hbm, pe_hbm, out_hbm,
           pe_v, ci_v, cr_v, idx_v, rows_v, ob_v, lat_s,
           gsem0, gsem1, osem0, osem1, csem):
    wid = lax.axis_index("s") * NC + lax.axis_index("c")
    base = wid * BPW
    gsems = (gsem0, gsem1)
    osems = (osem0, osem1)

    def gather_descs(buf, j):
        descs = []
        for off in _CHUNKS:
            iv = idx_v[j, pl.ds(off, L)]
            descs.append(pltpu.make_async_copy(
                lat_s.at[iv], rows_v.at[buf, pl.ds(off, L)], gsems[buf]))
        return descs

    def start_gather(buf, j):
        for d in gather_descs(buf, j):
            d.start()

    def wait_gather(buf, j):
        for d in gather_descs(buf, j):
            d.wait()

    def out_desc(buf, j):
        return pltpu.make_async_copy(ob_v.at[buf], out_hbm.at[base + j], osems[buf])

    def compute(buf, j):
        c = [cr_v[j, pl.ds(g * L, L)] for g in range(NG)]

        @pl.loop(0, SEQ)
        def _(s):
            for g in range(NG):
                sl = pl.ds(g * L, L)
                ob_v[buf, s, sl] = rows_v[buf, s, sl] + pe_v[s, sl] + c[g]

    sid = lax.axis_index("s")
    rpt = VOCAB // NS
    pltpu.sync_copy(lat_hbm.at[pl.ds(sid * rpt, rpt)],
                    lat_s.at[pl.ds(sid * rpt, rpt)])
    plsc.subcore_barrier()

    pltpu.sync_copy(pe_hbm, pe_v)
    pltpu.sync_copy(x_hbm.at[pl.ds(base, BPW)], idx_v)
    pltpu.sync_copy(cond_hbm.at[pl.ds(base, BPW)], ci_v)
    pltpu.async_copy(ct_hbm.at[ci_v], cr_v, csem).wait()

    start_gather(0, 0)

    @pl.loop(0, BPW // 2)
    def _(i):
        j0 = 2 * i

        start_gather(1, j0 + 1)
        wait_gather(0, j0)

        @pl.when(i >= 1)
        def _():
            out_desc(0, j0 - 2).wait()
        compute(0, j0)
        out_desc(0, j0).start()

        @pl.when(i < BPW // 2 - 1)
        def _():
            start_gather(0, j0 + 2)
        wait_gather(1, j0 + 1)

        @pl.when(i >= 1)
        def _():
            out_desc(1, j0 - 1).wait()
        compute(1, j0 + 1)
        out_desc(1, j0 + 1).start()

    out_desc(0, BPW - 2).wait()
    out_desc(1, BPW - 1).wait()


def kernel(x, cond, latent_table, cond_table, pe):
    return _embed(x, cond, latent_table, cond_table, pe[:SEQ])
